# Initial kernel scaffold; baseline (speedup 1.0000x reference)
import jax
import jax.numpy as jnp
from jax.experimental import pallas as pl


def kernel(nfeats, efeats, edge_index, Wm1, bm1, Wa1, ba1, Wm2, bm2, Wa2, ba2, Wp, bp):
    raise NotImplementedError("write your pallas kernel here")


# speedup vs baseline: 2.4775x; 2.4775x over previous
import functools
import jax
import jax.numpy as jnp
from jax import lax
from jax.experimental import pallas as pl
from jax.experimental.pallas import tpu as pltpu
from jax.experimental.pallas import tpu_sc as plsc

N = 10000
E = 320000
DIN = 128
EDIM = 16
DOUT = 128
NCLS = 11

NC = 2
NS = 16
NW = NC * NS
CH = 128
CHUNKS = 79
EPT = CH * CHUNKS
EP = EPT * NW
NP = 10240
RPT = NP // NS

_MESH = plsc.VectorSubcoreMesh(core_axis_name="c", subcore_axis_name="s")
_F32 = jnp.float32
_HIGH = lax.Precision.HIGHEST


def _dot(a, b):
    return jnp.dot(a, b, preferred_element_type=_F32, precision=_HIGH)


def _edge_msg_body(e_ref, w_ref, b_ref, o_ref):
    o_ref[...] = _dot(e_ref[...], w_ref[...]) + b_ref[...]


def _node_msg_body(h_ref, w_ref, o_ref):
    nb = h_ref.shape[0]
    hw = _dot(h_ref[...], w_ref[...])
    col = lax.broadcasted_iota(jnp.int32, (nb, DIN), 1)
    wide = jnp.concatenate(
        [hw, jnp.zeros((nb, DIN - EDIM), _F32)], axis=1)
    o_ref[...] = jnp.where(col == EDIM, 1.0, wide)


def _layer1_apply_body(h0_ref, s1_ref, wa1t_ref, wa1b_ref, ba1_ref,
                       wp2_ref, wq2_ref, bm2_ref, h1_ref, p_ref, qb_ref):
    acc = s1_ref[0] + s1_ref[1]
    cnt = acc[:, EDIM]
    inv = 1.0 / jnp.maximum(cnt, 1.0)
    hn1 = acc[:, :EDIM] * inv[:, None]
    h1 = jax.nn.relu(_dot(h0_ref[...], wa1t_ref[...]) +
                     _dot(hn1, wa1b_ref[...]) + ba1_ref[...])
    h1_ref[...] = h1
    p_ref[...] = _dot(h1, wp2_ref[...])
    qb_ref[...] = _dot(h1, wq2_ref[...]) + bm2_ref[...]


def _layer2_apply_body(h1_ref, s1_ref, s2_ref, qb_ref, wa2t_ref, wa2b_ref,
                       ba2_ref, wpt_ref, wpb_ref, bp_ref, ab_ref):
    nb = h1_ref.shape[0]
    cnt = s1_ref[0][:, EDIM] + s1_ref[1][:, EDIM]
    inv = 1.0 / jnp.maximum(cnt, 1.0)
    gate = jnp.minimum(cnt, 1.0)
    s2t = s2_ref[0] + s2_ref[1]
    hn2 = s2t * inv[:, None] + qb_ref[...] * gate[:, None]
    h2 = jax.nn.relu(_dot(h1_ref[...], wa2t_ref[...]) +
                     _dot(hn2, wa2b_ref[...]) + ba2_ref[...])
    a = _dot(h2, wpt_ref[...]) + bp_ref[...]
    b = _dot(h2, wpb_ref[...])
    ab_ref[...] = jnp.concatenate(
        [a, b, jnp.zeros((nb, DIN - 2 * EDIM), _F32)], axis=1)


@functools.partial(
    pl.kernel,
    out_type=jax.ShapeDtypeStruct((NC * NP, DIN), _F32),
    mesh=_MESH,
    scratch_types=[
        pltpu.VMEM((CH,), jnp.int32),
        pltpu.VMEM((CH,), jnp.int32),
        pltpu.VMEM((CH, DIN), _F32),
        pltpu.VMEM((CH, EDIM), _F32),
        pltpu.VMEM_SHARED((NP, DIN), _F32),
    ],
)
def _sc_layer1(hc_hbm, ew_hbm, src_hbm, dst_hbm, s1_hbm,
               sidx, didx, pbuf, ebuf, acc_sp):
    cid = lax.axis_index("c")
    sid = lax.axis_index("s")
    wid = cid * NS + sid
    zero = jnp.zeros((16,), _F32)
    for r in range(CH):
        for c in range(DIN // 16):
            pbuf[r, pl.ds(c * 16, 16)] = zero
    row0 = sid * RPT
    for r in range(RPT // CH):
        pltpu.sync_copy(pbuf, acc_sp.at[pl.ds(row0 + r * CH, CH)])
    plsc.subcore_barrier()

    ebase = wid * EPT

    def body(k, carry):
        off = ebase + k * CH
        pltpu.sync_copy(src_hbm.at[pl.ds(off, CH)], sidx)
        pltpu.sync_copy(dst_hbm.at[pl.ds(off, CH)], didx)
        pltpu.sync_copy(hc_hbm.at[sidx], pbuf)
        pltpu.sync_copy(ew_hbm.at[pl.ds(off, CH)], ebuf)
        for r in range(CH):
            pbuf[r, pl.ds(0, EDIM)] = pbuf[r, pl.ds(0, EDIM)] + ebuf[r, :]
        pltpu.sync_copy(pbuf, acc_sp.at[didx], add=True)
        return carry

    lax.fori_loop(0, CHUNKS, body, 0)
    plsc.subcore_barrier()
    obase = cid * NP + row0
    for r in range(RPT // CH):
        pltpu.sync_copy(acc_sp.at[pl.ds(row0 + r * CH, CH)], pbuf)
        pltpu.sync_copy(pbuf, s1_hbm.at[pl.ds(obase + r * CH, CH)])


@functools.partial(
    pl.kernel,
    out_type=jax.ShapeDtypeStruct((NC * NP, DOUT), _F32),
    mesh=_MESH,
    scratch_types=[
        pltpu.VMEM((CH,), jnp.int32),
        pltpu.VMEM((CH,), jnp.int32),
        pltpu.VMEM((CH, DOUT), _F32),
        pltpu.VMEM_SHARED((NP, DOUT), _F32),
    ],
)
def _sc_layer2(p_hbm, src_hbm, dst_hbm, s2_hbm, sidx, didx, pbuf, acc_sp):
    cid = lax.axis_index("c")
    sid = lax.axis_index("s")
    wid = cid * NS + sid
    zero = jnp.zeros((16,), _F32)
    for r in range(CH):
        for c in range(DOUT // 16):
            pbuf[r, pl.ds(c * 16, 16)] = zero
    row0 = sid * RPT
    for r in range(RPT // CH):
        pltpu.sync_copy(pbuf, acc_sp.at[pl.ds(row0 + r * CH, CH)])
    plsc.subcore_barrier()

    ebase = wid * EPT

    def body(k, carry):
        off = ebase + k * CH
        pltpu.sync_copy(src_hbm.at[pl.ds(off, CH)], sidx)
        pltpu.sync_copy(dst_hbm.at[pl.ds(off, CH)], didx)
        pltpu.sync_copy(p_hbm.at[sidx], pbuf)
        pltpu.sync_copy(pbuf, acc_sp.at[didx], add=True)
        return carry

    lax.fori_loop(0, CHUNKS, body, 0)
    plsc.subcore_barrier()
    obase = cid * NP + row0
    for r in range(RPT // CH):
        pltpu.sync_copy(acc_sp.at[pl.ds(row0 + r * CH, CH)], pbuf)
        pltpu.sync_copy(pbuf, s2_hbm.at[pl.ds(obase + r * CH, CH)])


@functools.partial(
    pl.kernel,
    out_type=jax.ShapeDtypeStruct((EP, EDIM), _F32),
    mesh=_MESH,
    scratch_types=[
        pltpu.VMEM((CH,), jnp.int32),
        pltpu.VMEM((CH,), jnp.int32),
        pltpu.VMEM((CH, DIN), _F32),
        pltpu.VMEM((CH, DIN), _F32),
        pltpu.VMEM((CH, EDIM), _F32),
    ],
)
def _sc_score(ab_hbm, src_hbm, dst_hbm, out_hbm,
              sidx, didx, ubuf, vbuf, obuf):
    cid = lax.axis_index("c")
    sid = lax.axis_index("s")
    wid = cid * NS + sid
    ebase = wid * EPT

    def body(k, carry):
        off = ebase + k * CH
        pltpu.sync_copy(src_hbm.at[pl.ds(off, CH)], sidx)
        pltpu.sync_copy(dst_hbm.at[pl.ds(off, CH)], didx)
        pltpu.sync_copy(ab_hbm.at[sidx], ubuf)
        pltpu.sync_copy(ab_hbm.at[didx], vbuf)
        for r in range(CH):
            obuf[r, :] = ubuf[r, pl.ds(0, EDIM)] + vbuf[r, pl.ds(EDIM, EDIM)]
        pltpu.sync_copy(obuf, out_hbm.at[pl.ds(off, CH)])
        return carry

    lax.fori_loop(0, CHUNKS, body, 0)


def kernel(nfeats, efeats, edge_index, Wm1, bm1, Wa1, ba1,
           Wm2, bm2, Wa2, ba2, Wp, bp):
    h0 = nfeats.reshape(N, DIN)
    h0p = jnp.pad(h0, ((0, NP - N), (0, 0)))
    e0 = efeats.reshape(E, EDIM)
    e0p = jnp.pad(e0, ((0, EP - E), (0, 0)))
    srcp = jnp.concatenate(
        [edge_index[0], jnp.full((EP - E,), N, jnp.int32)])
    dstp = jnp.concatenate(
        [edge_index[1], jnp.full((EP - E,), N, jnp.int32)])

    wm1t, wm1b = Wm1[:DIN], Wm1[DIN:]
    wa1t, wa1b = Wa1[:DIN], Wa1[DIN:]
    wp2 = Wm2[:EDIM] + 0.5 * Wm2[EDIM:]
    wq2 = 0.5 * Wm2[EDIM:]
    wa2t, wa2b = Wa2[:EDIM], Wa2[EDIM:]
    wpt = jnp.pad(Wp[:DOUT], ((0, 0), (0, 16 - NCLS)))
    wpb = jnp.pad(Wp[DOUT:], ((0, 0), (0, 16 - NCLS)))
    bpp = jnp.pad(bp, (0, 16 - NCLS)).reshape(1, 16)
    bm1r = bm1.reshape(1, EDIM)
    ba1r = ba1.reshape(1, EDIM)
    bm2r = bm2.reshape(1, DOUT)
    ba2r = ba2.reshape(1, DOUT)

    be = 2048
    ew = pl.pallas_call(
        _edge_msg_body,
        grid=(EP // be,),
        in_specs=[pl.BlockSpec((be, EDIM), lambda i: (i, 0)),
                  pl.BlockSpec((EDIM, EDIM), lambda i: (0, 0)),
                  pl.BlockSpec((1, EDIM), lambda i: (0, 0))],
        out_specs=pl.BlockSpec((be, EDIM), lambda i: (i, 0)),
        out_shape=jax.ShapeDtypeStruct((EP, EDIM), _F32),
    )(e0p, wm1b, bm1r)

    nb = 1280
    hc = pl.pallas_call(
        _node_msg_body,
        grid=(NP // nb,),
        in_specs=[pl.BlockSpec((nb, DIN), lambda i: (i, 0)),
                  pl.BlockSpec((DIN, EDIM), lambda i: (0, 0))],
        out_specs=pl.BlockSpec((nb, DIN), lambda i: (i, 0)),
        out_shape=jax.ShapeDtypeStruct((NP, DIN), _F32),
    )(h0p, wm1t)

    s1 = _sc_layer1(hc, ew, srcp, dstp).reshape(NC, NP, DIN)

    h1, pmat, qb = pl.pallas_call(
        _layer1_apply_body,
        grid=(NP // nb,),
        in_specs=[pl.BlockSpec((nb, DIN), lambda i: (i, 0)),
                  pl.BlockSpec((NC, nb, DIN), lambda i: (0, i, 0)),
                  pl.BlockSpec((DIN, EDIM), lambda i: (0, 0)),
                  pl.BlockSpec((EDIM, EDIM), lambda i: (0, 0)),
                  pl.BlockSpec((1, EDIM), lambda i: (0, 0)),
                  pl.BlockSpec((EDIM, DOUT), lambda i: (0, 0)),
                  pl.BlockSpec((EDIM, DOUT), lambda i: (0, 0)),
                  pl.BlockSpec((1, DOUT), lambda i: (0, 0))],
        out_specs=[pl.BlockSpec((nb, EDIM), lambda i: (i, 0)),
                   pl.BlockSpec((nb, DOUT), lambda i: (i, 0)),
                   pl.BlockSpec((nb, DOUT), lambda i: (i, 0))],
        out_shape=[jax.ShapeDtypeStruct((NP, EDIM), _F32),
                   jax.ShapeDtypeStruct((NP, DOUT), _F32),
                   jax.ShapeDtypeStruct((NP, DOUT), _F32)],
    )(h0p, s1, wa1t, wa1b, ba1r, wp2, wq2, bm2r)

    s2 = _sc_layer2(pmat, srcp, dstp).reshape(NC, NP, DOUT)

    ab = pl.pallas_call(
        _layer2_apply_body,
        grid=(NP // nb,),
        in_specs=[pl.BlockSpec((nb, EDIM), lambda i: (i, 0)),
                  pl.BlockSpec((NC, nb, DIN), lambda i: (0, i, 0)),
                  pl.BlockSpec((NC, nb, DOUT), lambda i: (0, i, 0)),
                  pl.BlockSpec((nb, DOUT), lambda i: (i, 0)),
                  pl.BlockSpec((EDIM, DOUT), lambda i: (0, 0)),
                  pl.BlockSpec((DOUT, DOUT), lambda i: (0, 0)),
                  pl.BlockSpec((1, DOUT), lambda i: (0, 0)),
                  pl.BlockSpec((DOUT, 16), lambda i: (0, 0)),
                  pl.BlockSpec((DOUT, 16), lambda i: (0, 0)),
                  pl.BlockSpec((1, 16), lambda i: (0, 0))],
        out_specs=pl.BlockSpec((nb, DIN), lambda i: (i, 0)),
        out_shape=jax.ShapeDtypeStruct((NP, DIN), _F32),
    )(h1, s1, s2, qb, wa2t, wa2b, ba2r, wpt, wpb, bpp)

    score = _sc_score(ab, srcp, dstp)
    return score[:E, :NCLS]
